# baseline (speedup 1.0000x reference)
import functools

import jax
import jax.numpy as jnp
from jax import lax
from jax.experimental import pallas as pl
from jax.experimental.pallas import tpu as pltpu
from jax.experimental.pallas import tpu_sc as plsc

VOCAB = 32000
HIDDEN = 2048
BATCH = 2
SEQ = 4096
B = BATCH * SEQ
NC, NS = 2, 16
NW = NC * NS
BPW = B // NW
TPB = NW // BATCH
CHUNK = 8
NCHUNK = BPW // CHUNK
NBUF = 4


def _build_gather():
    mesh = plsc.VectorSubcoreMesh(core_axis_name="c", subcore_axis_name="s")

    @functools.partial(
        pl.kernel,
        mesh=mesh,
        out_type=jax.ShapeDtypeStruct((BATCH, SEQ, HIDDEN), jnp.float32),
        scratch_types=(
            [pltpu.VMEM((BPW,), jnp.int32)]
            + [pltpu.VMEM((CHUNK, HIDDEN), jnp.float32)] * NBUF
            + [pltpu.SemaphoreType.DMA] * (2 * NBUF)
        ),
    )
    def gather_kernel(ids_hbm, table_hbm, out_hbm, idx_v, *scratch):
        shared = scratch[:NBUF]
        gsems = scratch[NBUF:2 * NBUF]
        psems = scratch[2 * NBUF:]
        wid = lax.axis_index("s") * NC + lax.axis_index("c")
        bidx = wid // TPB
        row0 = (wid % TPB) * BPW
        pltpu.sync_copy(ids_hbm.at[bidx, pl.ds(row0, BPW)], idx_v)

        bufs = shared

        gathers = [None] * NBUF
        puts = [None] * NBUF
        for j in range(NCHUNK):
            b = j % NBUF
            if puts[b] is not None:
                puts[b].wait()
            cp = pltpu.make_async_copy(
                table_hbm.at[idx_v.at[pl.ds(j * CHUNK, CHUNK)]],
                bufs[b], gsems[b])
            cp.start()
            gathers[b] = cp
            if j >= 1:
                q = (j - 1) % NBUF
                gathers[q].wait()
                out_cp = pltpu.make_async_copy(
                    bufs[q],
                    out_hbm.at[bidx, pl.ds(row0 + (j - 1) * CHUNK, CHUNK)],
                    psems[q])
                out_cp.start()
                puts[q] = out_cp
        last = (NCHUNK - 1) % NBUF
        gathers[last].wait()
        out_cp = pltpu.make_async_copy(
            bufs[last],
            out_hbm.at[bidx, pl.ds(row0 + (NCHUNK - 1) * CHUNK, CHUNK)],
            psems[last])
        out_cp.start()
        puts[last] = out_cp
        for p in puts:
            if p is not None:
                p.wait()

    return gather_kernel


_gather = _build_gather()


def kernel(input_ids, cu_seq_lens, cu_batch_size, embed_table):
    hidden = _gather(input_ids.astype(jnp.int32), embed_table)
    return hidden, cu_seq_lens, cu_batch_size


# speedup vs baseline: 1.0244x; 1.0244x over previous
import functools

import jax
import jax.numpy as jnp
from jax import lax
from jax.experimental import pallas as pl
from jax.experimental.pallas import tpu as pltpu
from jax.experimental.pallas import tpu_sc as plsc

VOCAB = 32000
HIDDEN = 2048
BATCH = 2
SEQ = 4096
B = BATCH * SEQ
NC, NS = 2, 16
NW = NC * NS
BPW = B // NW
TPB = NW // BATCH
CHUNK = 16
NCHUNK = BPW // CHUNK
NGROUP = NCHUNK // 2


def _build_gather():
    mesh = plsc.VectorSubcoreMesh(core_axis_name="c", subcore_axis_name="s")

    @functools.partial(
        pl.kernel,
        mesh=mesh,
        out_type=jax.ShapeDtypeStruct((BATCH, SEQ, HIDDEN), jnp.float32),
        scratch_types=[
            pltpu.VMEM((BPW,), jnp.int32),
            pltpu.VMEM((CHUNK, HIDDEN), jnp.float32),
            pltpu.VMEM((CHUNK, HIDDEN), jnp.float32),
            pltpu.SemaphoreType.DMA,
            pltpu.SemaphoreType.DMA,
            pltpu.SemaphoreType.DMA,
            pltpu.SemaphoreType.DMA,
        ],
    )
    def gather_kernel(ids_hbm, table_hbm, out_hbm,
                      idx_v, buf0, buf1, g0, g1, p0, p1):
        wid = lax.axis_index("s") * NC + lax.axis_index("c")
        bidx = wid // TPB
        row0 = (wid % TPB) * BPW
        pltpu.sync_copy(ids_hbm.at[bidx, pl.ds(row0, BPW)], idx_v)

        bufs = (buf0, buf1)
        gsems = (g0, g1)
        psems = (p0, p1)

        def start_gather(j, b):
            cp = pltpu.make_async_copy(
                table_hbm.at[idx_v.at[pl.ds(j * CHUNK, CHUNK)]],
                bufs[b], gsems[b])
            cp.start()
            return cp

        def start_put(j, b):
            cp = pltpu.make_async_copy(
                bufs[b], out_hbm.at[bidx, pl.ds(row0 + j * CHUNK, CHUNK)],
                psems[b])
            cp.start()
            return cp

        start_gather(0, 0)
        start_gather(1, 1)
        pltpu.make_async_copy(table_hbm.at[idx_v.at[pl.ds(0, CHUNK)]],
                              buf0, gsems[0]).wait()
        start_put(0, 0)

        def group(g, carry):
            j0 = g * 2
            for b in (0, 1):
                j = j0 + b
                pltpu.make_async_copy(
                    bufs[b], out_hbm.at[bidx, pl.ds(row0, CHUNK)],
                    psems[b]).wait()
                start_gather(j, b)
                q = 1 - b
                pltpu.make_async_copy(
                    table_hbm.at[idx_v.at[pl.ds(0, CHUNK)]],
                    bufs[q], gsems[q]).wait()
                start_put(j - 1, q)
            return carry

        lax.fori_loop(1, NGROUP, group, 0)

        last = (NCHUNK - 1) % 2
        pltpu.make_async_copy(table_hbm.at[idx_v.at[pl.ds(0, CHUNK)]],
                              bufs[last], gsems[last]).wait()
        start_put(NCHUNK - 1, last)
        for b in (0, 1):
            pltpu.make_async_copy(
                bufs[b], out_hbm.at[bidx, pl.ds(row0, CHUNK)],
                psems[b]).wait()

    return gather_kernel


_gather = _build_gather()


def kernel(input_ids, cu_seq_lens, cu_batch_size, embed_table):
    hidden = _gather(input_ids.astype(jnp.int32), embed_table)
    return hidden, cu_seq_lens, cu_batch_size
